# baseline (speedup 1.0000x reference)
import numpy as np
import jax
import jax.numpy as jnp
from jax.experimental import pallas as pl
from jax.experimental.pallas import tpu as pltpu

_K0 = np.uint32(867802714)
_K1 = np.uint32(3762255628)
_K2 = np.uint32(int(_K0) ^ int(_K1) ^ 0x1BD11BDA)

_ROTS = ((13, 15, 26, 6), (17, 29, 16, 24))
_INJ = ((1, 2, 1), (2, 0, 2), (0, 1, 3), (1, 2, 4), (2, 0, 5))

_C_SMALL = (2.81022636e-08, 3.43273939e-07, -3.5233877e-06, -4.39150654e-06,
            0.00021858087, -0.00125372503, -0.00417768164, 0.246640727,
            1.50140941)
_C_BIG = (-0.000200214257, 0.000100950558, 0.00134934322, -0.00367342844,
          0.00573950773, -0.0076224613, 0.00943887047, 1.00167406,
          2.83297682)

_LO = np.float32(-0.99999994)
_SCALE = np.float32(1.0) - _LO
_SQRT2 = np.float32(np.sqrt(2.0))
_SQRT10 = np.float32(np.sqrt(10.0))
_COS_CONST = np.float32(8.0 * np.cos(1.2 * 3))
_HALF_LOG_2PI = np.float32(0.5 * np.log(2.0 * np.pi))


def _poly(coeffs, x):
    p = jnp.full_like(x, jnp.float32(coeffs[0]))
    for c in coeffs[1:]:
        p = p * x + jnp.float32(c)
    return p


def _pf_kernel(p_ref, w_ref, obs_ref, out_p_ref, out_w_ref):
    p = p_ref[...]
    w = w_ref[...]
    obs = obs_ref[...]
    b, n = p.shape

    row0 = jnp.uint32(pl.program_id(0)) * jnp.uint32(b)
    row = jax.lax.broadcasted_iota(jnp.uint32, (b, n), 0) + row0
    col = jax.lax.broadcasted_iota(jnp.uint32, (b, n), 1)
    ks = (jnp.uint32(_K0), jnp.uint32(_K1), jnp.uint32(_K2))
    x0 = jnp.full((b, n), _K0, dtype=jnp.uint32)
    x1 = row * jnp.uint32(n) + col + jnp.uint32(_K1)
    for g in range(5):
        for r in _ROTS[g % 2]:
            x0 = x0 + x1
            x1 = (x1 << jnp.uint32(r)) | (x1 >> jnp.uint32(32 - r))
            x1 = x0 ^ x1
        a, bb, c = _INJ[g]
        x0 = x0 + ks[a]
        x1 = x1 + ks[bb] + jnp.uint32(c)
    bits = x0 ^ x1

    fb = (bits >> jnp.uint32(9)) | jnp.uint32(0x3F800000)
    f = jax.lax.bitcast_convert_type(fb, jnp.float32) - jnp.float32(1.0)
    u = jnp.maximum(jnp.float32(_LO), f * jnp.float32(_SCALE) + jnp.float32(_LO))
    lw = -jnp.log1p(-(u * u))
    small = lw < jnp.float32(5.0)
    ps = _poly(_C_SMALL, lw - jnp.float32(2.5))
    pb = _poly(_C_BIG, jnp.sqrt(jnp.maximum(lw, jnp.float32(5.0))) - jnp.float32(3.0))
    z = jnp.float32(_SQRT2) * jnp.where(small, ps, pb) * u
    noise = z * jnp.float32(_SQRT10)

    mean = p / jnp.float32(2.0)
    mean = mean + jnp.float32(25.0) * (p / (p * p + jnp.float32(1.0)))
    mean = mean + jnp.float32(_COS_CONST)
    pn = mean + noise
    out_p_ref[...] = pn

    om = (pn * pn) / jnp.float32(20.0)
    d = obs - om
    log_prob = jnp.float32(-0.5) * (d * d) - jnp.float32(_HALF_LOG_2PI)
    wn = w * jnp.exp(log_prob)
    s = jnp.sum(wn, axis=1, keepdims=True)
    out_w_ref[...] = wn / s


def kernel(particles, particle_weights, observation, timestep_number):
    del timestep_number
    b, n, d = particles.shape
    p2 = particles.reshape(b, n)
    cores = 2
    br = b // cores
    pn, wn = pl.pallas_call(
        _pf_kernel,
        grid=(cores,),
        in_specs=[
            pl.BlockSpec((br, n), lambda i: (i, 0)),
            pl.BlockSpec((br, n), lambda i: (i, 0)),
            pl.BlockSpec((br, 1), lambda i: (i, 0)),
        ],
        out_specs=(
            pl.BlockSpec((br, n), lambda i: (i, 0)),
            pl.BlockSpec((br, n), lambda i: (i, 0)),
        ),
        out_shape=(
            jax.ShapeDtypeStruct((b, n), jnp.float32),
            jax.ShapeDtypeStruct((b, n), jnp.float32),
        ),
        compiler_params=pltpu.CompilerParams(
            dimension_semantics=("parallel",),
        ),
    )(p2, particle_weights, observation)
    return pn.reshape(b, n, d), wn


# speedup vs baseline: 1.1999x; 1.1999x over previous
import numpy as np
import jax
import jax.numpy as jnp
from jax.experimental import pallas as pl

_K0 = np.uint32(867802714)
_K1 = np.uint32(3762255628)

_COS_CONST = np.float32(8.0 * np.cos(1.2 * 3))
_HALF_LOG_2PI = np.float32(0.5 * np.log(2.0 * np.pi))


def _threefry_normal(n_elems: int) -> np.ndarray:
    old = np.seterr(over="ignore")
    try:
        ks = (_K0, _K1, np.uint32(int(_K0) ^ int(_K1) ^ 0x1BD11BDA))
        x0 = np.full(n_elems, ks[0], dtype=np.uint32)
        x1 = (np.arange(n_elems, dtype=np.uint32) + ks[1]).astype(np.uint32)
        rots = ((13, 15, 26, 6), (17, 29, 16, 24))
        inj = ((1, 2, 1), (2, 0, 2), (0, 1, 3), (1, 2, 4), (2, 0, 5))
        for g in range(5):
            for r in rots[g % 2]:
                x0 = (x0 + x1).astype(np.uint32)
                x1 = ((x1 << np.uint32(r)) | (x1 >> np.uint32(32 - r))).astype(np.uint32)
                x1 = (x0 ^ x1).astype(np.uint32)
            a, b, c = inj[g]
            x0 = (x0 + ks[a]).astype(np.uint32)
            x1 = (x1 + ks[b] + np.uint32(c)).astype(np.uint32)
        bits = (x0 ^ x1).astype(np.uint32)
        fb = ((bits >> np.uint32(9)) | np.uint32(0x3F800000)).view(np.float32)
        f = fb - np.float32(1.0)
        lo = np.float32(-0.99999994)
        u = np.maximum(lo, (f * (np.float32(1.0) - lo) + lo).astype(np.float32))
        w = (-np.log1p(-(u * u))).astype(np.float32)
        cs = (2.81022636e-08, 3.43273939e-07, -3.5233877e-06, -4.39150654e-06,
              0.00021858087, -0.00125372503, -0.00417768164, 0.246640727,
              1.50140941)
        cb = (-0.000200214257, 0.000100950558, 0.00134934322, -0.00367342844,
              0.00573950773, -0.0076224613, 0.00943887047, 1.00167406,
              2.83297682)

        def poly(c, x):
            p = np.full_like(x, np.float32(c[0]))
            for cc in c[1:]:
                p = (p * x + np.float32(cc)).astype(np.float32)
            return p

        ps = poly(cs, (w - np.float32(2.5)).astype(np.float32))
        pb = poly(cb, (np.sqrt(np.maximum(w, np.float32(5.0))) - np.float32(3.0)).astype(np.float32))
        z = (np.float32(np.sqrt(2.0)) * np.where(w < np.float32(5.0), ps, pb) * u).astype(np.float32)
        return z
    finally:
        np.seterr(**old)


def _pf_kernel(p_ref, w_ref, obs_ref, noise_ref, out_p_ref, out_w_ref):
    p = p_ref[...]
    w = w_ref[...]
    obs = obs_ref[...]

    pn = p * jnp.float32(0.5) + jnp.float32(25.0) * (p / (p * p + jnp.float32(1.0)))
    pn = pn + noise_ref[...]
    out_p_ref[...] = pn

    om = (pn * pn) / jnp.float32(20.0)
    d = obs - om
    log_prob = jnp.float32(-0.5) * (d * d) - jnp.float32(_HALF_LOG_2PI)
    wn = w * jnp.exp(log_prob)
    s = jnp.sum(wn, axis=1, keepdims=True)
    out_w_ref[...] = wn * (jnp.float32(1.0) / s)


def kernel(particles, particle_weights, observation, timestep_number):
    del timestep_number
    b, n, d = particles.shape
    p2 = particles.reshape(b, n)
    zn = _threefry_normal(b * n).reshape(b, n)
    addterm = (np.float32(np.sqrt(10.0)) * zn + _COS_CONST).astype(np.float32)
    pn, wn = pl.pallas_call(
        _pf_kernel,
        out_shape=(
            jax.ShapeDtypeStruct((b, n), jnp.float32),
            jax.ShapeDtypeStruct((b, n), jnp.float32),
        ),
    )(p2, particle_weights, observation, jnp.asarray(addterm))
    return pn.reshape(b, n, d), wn
